# baseline (speedup 1.0000x reference)
import functools
import jax
import jax.numpy as jnp
from jax import lax
from jax.experimental import pallas as pl
from jax.experimental.pallas import tpu as pltpu
from jax.experimental.pallas import tpu_sc as plsc

_TM = 256
_LANE = 128


def _router_tile(x_ref, w1_ref, b1_ref, w2_ref, b2_ref,
                 i1_ref, i2_ref, w1o_ref, w2o_ref):
    n_e = 8
    x = x_ref[...]
    xb = x.astype(jnp.bfloat16)
    h = jnp.dot(xb, w1_ref[...].astype(jnp.bfloat16),
                preferred_element_type=jnp.float32)
    h = h + b1_ref[...]
    h = h * jax.lax.logistic(h)
    rv = jnp.dot(h, w2_ref[...], preferred_element_type=jnp.float32)
    rv = rv + b2_ref[...]
    col = jax.lax.broadcasted_iota(jnp.int32, rv.shape, 1)
    neg = jnp.float32(-jnp.inf)
    rv = jnp.where(col < n_e, rv, neg)
    m1 = jnp.max(rv, axis=1, keepdims=True)
    i1 = jnp.min(jnp.where(rv == m1, col, _LANE), axis=1, keepdims=True)
    rv2 = jnp.where(col == i1, neg, rv)
    m2 = jnp.max(rv2, axis=1, keepdims=True)
    i2 = jnp.min(jnp.where(rv2 == m2, col, _LANE), axis=1, keepdims=True)
    e2 = jnp.exp(m2 - m1)
    w1v = 1.0 / (1.0 + e2)
    i1_ref[...] = i1
    i2_ref[...] = i2
    w1o_ref[...] = w1v
    w2o_ref[...] = e2 * w1v


def _run_router(x2, W1, b1, W2, b2):
    n_tok, d = x2.shape
    n_e = W2.shape[1]
    w2p = jnp.zeros((d, _LANE), W2.dtype).at[:, :n_e].set(W2)
    b2p = jnp.zeros((1, _LANE), b2.dtype).at[:, :n_e].set(b2)
    grid = (n_tok // _TM,)
    o = jax.ShapeDtypeStruct((n_tok, 1), jnp.int32)
    of = jax.ShapeDtypeStruct((n_tok, 1), jnp.float32)
    ospec = pl.BlockSpec((_TM, 1), lambda t: (t, 0))
    return pl.pallas_call(
        _router_tile,
        grid=grid,
        in_specs=[
            pl.BlockSpec((_TM, d), lambda t: (t, 0)),
            pl.BlockSpec((d, d), lambda t: (0, 0)),
            pl.BlockSpec((1, d), lambda t: (0, 0)),
            pl.BlockSpec((d, _LANE), lambda t: (0, 0)),
            pl.BlockSpec((1, _LANE), lambda t: (0, 0)),
        ],
        out_specs=[ospec, ospec, ospec, ospec],
        out_shape=[o, o, of, of],
        compiler_params=pltpu.CompilerParams(
            dimension_semantics=("arbitrary",),
        ),
    )(x2, W1, b1.reshape(1, d), w2p, b2p)


def _jnp_binning(eid, wt, n_tok, n_e, n_pad):
    s_tot = eid.shape[0]
    cnt = jnp.bincount(eid, length=n_e)
    padded = (cnt + _TM - 1) // _TM * _TM
    off_pad = jnp.cumsum(padded) - padded
    off_sort = jnp.cumsum(cnt) - cnt
    order = jnp.argsort(eid, stable=True)
    inv = jnp.zeros((s_tot,), jnp.int32).at[order].set(
        jnp.arange(s_tot, dtype=jnp.int32))
    pos = off_pad[eid] + inv - off_sort[eid]
    src = jnp.zeros((n_pad,), jnp.int32).at[pos].set(
        jnp.arange(s_tot, dtype=jnp.int32) % n_tok)
    swt = jnp.zeros((n_pad,), jnp.float32).at[pos].set(wt)
    ends = off_pad + padded
    tile_start = jnp.arange(128, dtype=jnp.int32) * _TM
    teid = jnp.minimum(
        jnp.sum(tile_start[:, None] >= ends[None, :], axis=1), n_e - 1
    ).astype(jnp.int32)
    return src, swt, pos, teid


def _group_tile(teid_ref, xs_ref, ew_ref, eb_ref, swt_ref, ys_ref):
    xsb = xs_ref[...].astype(jnp.bfloat16)
    eo = jnp.dot(xsb, ew_ref[0].astype(jnp.bfloat16),
                 preferred_element_type=jnp.float32)
    ys_ref[...] = (eo + eb_ref[0]) * swt_ref[...]


def _run_group(xs, expert_W, expert_b, swt, teid, n_tiles):
    n_e, d, _ = expert_W.shape
    grid_spec = pltpu.PrefetchScalarGridSpec(
        num_scalar_prefetch=1,
        grid=(n_tiles,),
        in_specs=[
            pl.BlockSpec((_TM, d), lambda j, tr: (j, 0)),
            pl.BlockSpec((1, d, d), lambda j, tr: (tr[j], 0, 0)),
            pl.BlockSpec((1, 1, d), lambda j, tr: (tr[j], 0, 0)),
            pl.BlockSpec((_TM, 1), lambda j, tr: (j, 0)),
        ],
        out_specs=pl.BlockSpec((_TM, d), lambda j, tr: (j, 0)),
    )
    return pl.pallas_call(
        _group_tile,
        grid_spec=grid_spec,
        out_shape=jax.ShapeDtypeStruct((n_tiles * _TM, d), jnp.float32),
        compiler_params=pltpu.CompilerParams(
            dimension_semantics=("arbitrary",),
        ),
    )(teid, xs, expert_W, expert_b.reshape(n_e, 1, d),
      swt.reshape(-1, 1)[: n_tiles * _TM])


def kernel(x, W1, b1, W2, b2, expert_W, expert_b):
    n_b, n_f, d = x.shape
    n_e = expert_W.shape[0]
    n_tok = n_b * n_f
    s_tot = 2 * n_tok
    n_pad = s_tot + n_e * _TM
    n_tiles = n_pad // _TM
    x2 = x.reshape(n_tok, d)

    i1, i2, w1v, w2v = _run_router(x2, W1, b1, W2, b2)
    eid = jnp.concatenate([i1.reshape(n_tok), i2.reshape(n_tok)])
    wt = jnp.concatenate([w1v.reshape(n_tok), w2v.reshape(n_tok)])

    src, swt, pos, teid = _jnp_binning(eid, wt, n_tok, n_e, n_pad)
    xs = x2[src]
    ys = _run_group(xs, expert_W, expert_b, swt, teid, n_tiles)
    out = x2 + ys[pos[:n_tok]] + ys[pos[n_tok:]]
    return out.reshape(n_b, n_f, d)


# speedup vs baseline: 3.1581x; 3.1581x over previous
import jax
import jax.numpy as jnp
from jax.experimental import pallas as pl
from jax.experimental.pallas import tpu as pltpu

_TM = 256
_LANE = 128


def _moe_tile(x_ref, w1_ref, b1_ref, w2_ref, b2_ref, ew_ref, eb_ref, o_ref):
    n_e = ew_ref.shape[0]
    x = x_ref[...]
    xb = x.astype(jnp.bfloat16)
    h = jnp.dot(xb, w1_ref[...].astype(jnp.bfloat16),
                preferred_element_type=jnp.float32)
    h = h + b1_ref[...]
    h = h * jax.lax.logistic(h)
    rv = jnp.dot(h, w2_ref[...], preferred_element_type=jnp.float32)
    rv = rv + b2_ref[...]
    col = jax.lax.broadcasted_iota(jnp.int32, rv.shape, 1)
    neg = jnp.float32(-jnp.inf)
    rv = jnp.where(col < n_e, rv, neg)
    m1 = jnp.max(rv, axis=1, keepdims=True)
    i1 = jnp.min(jnp.where(rv == m1, col, _LANE), axis=1, keepdims=True)
    sel1 = col == i1
    rv2 = jnp.where(sel1, neg, rv)
    m2 = jnp.max(rv2, axis=1, keepdims=True)
    i2 = jnp.min(jnp.where(rv2 == m2, col, _LANE), axis=1, keepdims=True)
    sel2 = col == i2
    e2 = jnp.exp(m2 - m1)
    w1v = 1.0 / (1.0 + e2)
    w2v = e2 * w1v
    wd = jnp.where(sel1, w1v, 0.0) + jnp.where(sel2, w2v, 0.0)
    acc = x
    for e in range(n_e):
        eo = jnp.dot(xb, ew_ref[e].astype(jnp.bfloat16),
                     preferred_element_type=jnp.float32)
        eo = eo + eb_ref[e][None, :]
        acc = acc + eo * wd[:, e][:, None]
    o_ref[...] = acc


def kernel(x, W1, b1, W2, b2, expert_W, expert_b):
    n_b, n_f, d = x.shape
    n_e = expert_W.shape[0]
    n_tok = n_b * n_f
    x2 = x.reshape(n_tok, d)
    w2p = jnp.zeros((d, _LANE), W2.dtype).at[:, :n_e].set(W2)
    b2p = jnp.zeros((1, _LANE), b2.dtype).at[:, :n_e].set(b2)
    grid = (n_tok // _TM,)
    out = pl.pallas_call(
        _moe_tile,
        grid=grid,
        in_specs=[
            pl.BlockSpec((_TM, d), lambda t: (t, 0)),
            pl.BlockSpec((d, d), lambda t: (0, 0)),
            pl.BlockSpec((1, d), lambda t: (0, 0)),
            pl.BlockSpec((d, _LANE), lambda t: (0, 0)),
            pl.BlockSpec((1, _LANE), lambda t: (0, 0)),
            pl.BlockSpec((n_e, d, d), lambda t: (0, 0, 0)),
            pl.BlockSpec((n_e, d), lambda t: (0, 0)),
        ],
        out_specs=pl.BlockSpec((_TM, d), lambda t: (t, 0)),
        out_shape=jax.ShapeDtypeStruct((n_tok, d), jnp.float32),
        compiler_params=pltpu.CompilerParams(
            dimension_semantics=("arbitrary",),
        ),
    )(x2, W1, b1.reshape(1, d), w2p, b2p, expert_W, expert_b)
    return out.reshape(n_b, n_f, d)


# speedup vs baseline: 3.2838x; 1.0398x over previous
import jax
import jax.numpy as jnp
from jax.experimental import pallas as pl
from jax.experimental.pallas import tpu as pltpu

_TM = 512
_LANE = 128


def _moe_tile(x_ref, w1_ref, b1_ref, w2_ref, b2_ref, ew_ref, eb_ref, o_ref):
    n_e = ew_ref.shape[0]
    x = x_ref[...]
    xb = x.astype(jnp.bfloat16)
    h = jnp.dot(xb, w1_ref[...].astype(jnp.bfloat16),
                preferred_element_type=jnp.float32)
    h = h + b1_ref[...]
    h = h * jax.lax.logistic(h)
    rv = jnp.dot(h, w2_ref[...], preferred_element_type=jnp.float32)
    rv = rv + b2_ref[...]
    col = jax.lax.broadcasted_iota(jnp.int32, rv.shape, 1)
    neg = jnp.float32(-jnp.inf)
    rv = jnp.where(col < n_e, rv, neg)
    m1 = jnp.max(rv, axis=1, keepdims=True)
    i1 = jnp.min(jnp.where(rv == m1, col, _LANE), axis=1, keepdims=True)
    sel1 = col == i1
    rv2 = jnp.where(sel1, neg, rv)
    m2 = jnp.max(rv2, axis=1, keepdims=True)
    i2 = jnp.min(jnp.where(rv2 == m2, col, _LANE), axis=1, keepdims=True)
    sel2 = col == i2
    e2 = jnp.exp(m2 - m1)
    w1v = 1.0 / (1.0 + e2)
    w2v = e2 * w1v
    wd = jnp.where(sel1, w1v, 0.0) + jnp.where(sel2, w2v, 0.0)
    acc = x
    for e in range(n_e):
        eo = jnp.dot(xb, ew_ref[e].astype(jnp.bfloat16),
                     preferred_element_type=jnp.float32)
        eo = eo + eb_ref[e][None, :]
        acc = acc + eo * wd[:, e][:, None]
    o_ref[...] = acc


def kernel(x, W1, b1, W2, b2, expert_W, expert_b):
    n_b, n_f, d = x.shape
    n_e = expert_W.shape[0]
    n_tok = n_b * n_f
    x2 = x.reshape(n_tok, d)
    w2p = jnp.zeros((d, _LANE), W2.dtype).at[:, :n_e].set(W2)
    b2p = jnp.zeros((1, _LANE), b2.dtype).at[:, :n_e].set(b2)
    grid = (n_tok // _TM,)
    out = pl.pallas_call(
        _moe_tile,
        grid=grid,
        in_specs=[
            pl.BlockSpec((_TM, d), lambda t: (t, 0)),
            pl.BlockSpec((d, d), lambda t: (0, 0)),
            pl.BlockSpec((1, d), lambda t: (0, 0)),
            pl.BlockSpec((d, _LANE), lambda t: (0, 0)),
            pl.BlockSpec((1, _LANE), lambda t: (0, 0)),
            pl.BlockSpec((n_e, d, d), lambda t: (0, 0, 0)),
            pl.BlockSpec((n_e, d), lambda t: (0, 0)),
        ],
        out_specs=pl.BlockSpec((_TM, d), lambda t: (t, 0)),
        out_shape=jax.ShapeDtypeStruct((n_tok, d), jnp.float32),
        compiler_params=pltpu.CompilerParams(
            dimension_semantics=("arbitrary",),
        ),
    )(x2, W1, b1.reshape(1, d), w2p, b2p, expert_W, expert_b)
    return out.reshape(n_b, n_f, d)
